# baseline (speedup 1.0000x reference)
import functools

import jax
import jax.numpy as jnp
from jax import lax
from jax.experimental import pallas as pl
from jax.experimental.pallas import tpu as pltpu
from jax.experimental.pallas import tpu_sc as plsc

N_GRAPHS = 128

NC, NS = 2, 16
NW = NC * NS
CHUNK = 128


def _sc_segment_sum(h, src2d, dst2d, n_nodes, agg_rows, ch_per_w):
    d = h.shape[1]
    zc = agg_rows // NS // CHUNK
    out_rows = agg_rows // NS
    gs = 8
    n_gpairs = ch_per_w // (2 * gs)
    n_pairs = gs // 2

    mesh = plsc.VectorSubcoreMesh(core_axis_name="c", subcore_axis_name="s")

    @functools.partial(
        pl.kernel,
        out_type=jax.ShapeDtypeStruct((NC, agg_rows, d), jnp.float32),
        mesh=mesh,
        scratch_types=[
            pltpu.VMEM((gs, CHUNK), jnp.int32),
            pltpu.VMEM((gs, CHUNK), jnp.int32),
            pltpu.VMEM((gs, CHUNK), jnp.int32),
            pltpu.VMEM((gs, CHUNK), jnp.int32),
            pltpu.VMEM((CHUNK, d), jnp.float32),
            pltpu.VMEM((CHUNK, d), jnp.float32),
            pltpu.VMEM_SHARED((agg_rows, d), jnp.float32),
            pltpu.SemaphoreType.DMA,
            pltpu.SemaphoreType.DMA,
            pltpu.SemaphoreType.DMA,
        ],
    )
    def kern(h_hbm, src_hbm, dst_hbm, out_hbm, src_a, dst_a, src_b, dst_b,
             rows_a, rows_b, agg_s, sem_a, sem_b, sem_i):
        cid = lax.axis_index("c")
        sid = lax.axis_index("s")
        wid = sid * NC + cid
        base = wid * ch_per_w

        def stage_start(g, sv, dv):
            pltpu.async_copy(src_hbm.at[pl.ds(base + g * gs, gs)], sv, sem_i)
            pltpu.async_copy(dst_hbm.at[pl.ds(base + g * gs, gs)], dv, sem_i)

        def stage_wait(g, sv, dv):
            pltpu.make_async_copy(
                src_hbm.at[pl.ds(base + g * gs, gs)], sv, sem_i).wait()
            pltpu.make_async_copy(
                dst_hbm.at[pl.ds(base + g * gs, gs)], dv, sem_i).wait()

        with jax.named_scope("agg_zero"):
            def zero_body(i, _):
                rows_a[i // (d // 16), pl.ds((i % (d // 16)) * 16, 16)] = (
                    jnp.zeros((16,), jnp.float32))
                return _
            lax.fori_loop(0, CHUNK * (d // 16), zero_body, None)
            stage_start(0, src_a, dst_a)
            for m in range(zc):
                pltpu.async_copy(
                    rows_a,
                    agg_s.at[pl.ds(sid * (zc * CHUNK) + m * CHUNK, CHUNK)],
                    sem_a)
            for m in range(zc):
                pltpu.make_async_copy(
                    rows_a,
                    agg_s.at[pl.ds(sid * (zc * CHUNK) + m * CHUNK, CHUNK)],
                    sem_a).wait()
            stage_wait(0, src_a, dst_a)

        plsc.subcore_barrier()

        def start(idx, rows, sem):
            pltpu.async_copy(h_hbm.at[idx], rows, sem)

        def wait(idx, rows, sem):
            pltpu.make_async_copy(h_hbm.at[idx], rows, sem).wait()

        def scat(idx, rows):
            pltpu.sync_copy(rows, agg_s.at[idx], add=True)

        def run_group(sv, dv):
            start(sv.at[0], rows_a, sem_a)

            def pair_body(jj, _):
                j0 = 2 * jj
                start(sv.at[j0 + 1], rows_b, sem_b)
                wait(sv.at[j0], rows_a, sem_a)
                scat(dv.at[j0], rows_a)

                @pl.when(jj < n_pairs - 1)
                def _():
                    start(sv.at[j0 + 2], rows_a, sem_a)

                wait(sv.at[j0 + 1], rows_b, sem_b)
                scat(dv.at[j0 + 1], rows_b)
                return _
            lax.fori_loop(0, n_pairs, pair_body, None)

        def gpair_body(gg, _):
            g0 = 2 * gg

            @pl.when(gg > 0)
            def _():
                stage_wait(g0, src_a, dst_a)

            stage_start(g0 + 1, src_b, dst_b)
            run_group(src_a, dst_a)
            stage_wait(g0 + 1, src_b, dst_b)

            @pl.when(gg < n_gpairs - 1)
            def _():
                stage_start(g0 + 2, src_a, dst_a)

            run_group(src_b, dst_b)
            return _
        with jax.named_scope("agg_edges"):
            lax.fori_loop(0, n_gpairs, gpair_body, None)

        plsc.subcore_barrier()

        with jax.named_scope("agg_writeback"):
            pltpu.sync_copy(agg_s.at[pl.ds(sid * out_rows, out_rows)],
                            out_hbm.at[cid, pl.ds(sid * out_rows, out_rows)])

    return kern(h, src2d, dst2d)[:, :n_nodes, :]


def _tc_layer(p0, p1, h, w_rel, w_root, bias, relu):
    n, d = h.shape
    blk = 2000
    grid = n // blk

    def body(p0_ref, p1_ref, h_ref, wr_ref, wt_ref, b_ref, o_ref):
        agg = p0_ref[...] + p1_ref[...]
        acc = jnp.dot(agg, wr_ref[...], preferred_element_type=jnp.float32)
        acc = acc + jnp.dot(h_ref[...], wt_ref[...],
                            preferred_element_type=jnp.float32)
        acc = acc + b_ref[...]
        o_ref[...] = jnp.maximum(acc, 0.0) if relu else acc

    row_spec = pl.BlockSpec((blk, d), lambda i: (i, 0))
    full = pl.BlockSpec((d, d), lambda i: (0, 0))
    return pl.pallas_call(
        body,
        grid=(grid,),
        in_specs=[row_spec, row_spec, row_spec, full, full,
                  pl.BlockSpec((1, d), lambda i: (0, 0))],
        out_specs=row_spec,
        out_shape=jax.ShapeDtypeStruct((n, d), jnp.float32),
    )(p0, p1, h, w_rel, w_root, bias)


def _tc_pool_linear(h, batch2d, w_lin, b_lin2d):
    n, d = h.shape
    g = N_GRAPHS
    c = w_lin.shape[1]
    blk = 2000
    grid = n // blk

    def body(h_ref, bt_ref, wl_ref, bl_ref, o_ref, acc_ref, cnt_ref):
        i = pl.program_id(0)

        @pl.when(i == 0)
        def _():
            acc_ref[...] = jnp.zeros_like(acc_ref)
            cnt_ref[...] = jnp.zeros_like(cnt_ref)

        onehot = (bt_ref[...] == lax.broadcasted_iota(jnp.int32, (blk, g), 1)
                  ).astype(jnp.float32)
        acc_ref[...] += lax.dot_general(
            onehot, h_ref[...], (((0,), (0,)), ((), ())),
            preferred_element_type=jnp.float32)
        cnt_ref[...] += lax.dot_general(
            onehot, jnp.ones((blk, g), jnp.float32), (((0,), (0,)), ((), ())),
            preferred_element_type=jnp.float32)

        @pl.when(i == grid - 1)
        def _():
            pooled = acc_ref[...] / jnp.maximum(cnt_ref[...], 1.0)
            o_ref[...] = jnp.dot(pooled, wl_ref[...],
                                 preferred_element_type=jnp.float32) + bl_ref[...]

    return pl.pallas_call(
        body,
        grid=(grid,),
        in_specs=[
            pl.BlockSpec((blk, d), lambda i: (i, 0)),
            pl.BlockSpec((blk, g), lambda i: (i, 0)),
            pl.BlockSpec((d, c), lambda i: (0, 0)),
            pl.BlockSpec((1, c), lambda i: (0, 0)),
        ],
        out_specs=pl.BlockSpec((g, c), lambda i: (0, 0)),
        out_shape=jax.ShapeDtypeStruct((g, c), jnp.float32),
        scratch_shapes=[
            pltpu.VMEM((g, g), jnp.float32),
            pltpu.VMEM((g, g), jnp.float32),
        ],
    )(h, batch2d, w_lin, b_lin2d)


def kernel(x, edge_index, batch, W_rel, W_root, b, W_lin, b_lin):
    n, d = x.shape
    e = edge_index.shape[1]

    ch_per_w = -(-(-(-e // (NW * CHUNK))) // 16) * 16
    e_pad = NW * CHUNK * ch_per_w
    agg_rows = -(-(n + 1) // (NS * CHUNK)) * (NS * CHUNK)

    src = edge_index[0].astype(jnp.int32)
    dst = edge_index[1].astype(jnp.int32)
    pad = e_pad - e
    pad_src = jnp.arange(pad, dtype=jnp.int32) % n
    pad_dst = n + jnp.arange(pad, dtype=jnp.int32) % (agg_rows - n)
    src2d = jnp.concatenate([src, pad_src]).reshape(-1, CHUNK)
    dst2d = jnp.concatenate([dst, pad_dst]).reshape(-1, CHUNK)

    h = x
    for i in range(8):
        p = _sc_segment_sum(h, src2d, dst2d, n, agg_rows, ch_per_w)
        h = _tc_layer(p[0], p[1], h, W_rel[i], W_root[i], b[i][None, :],
                      relu=(i < 7))

    batch2d = jnp.broadcast_to(batch.astype(jnp.int32)[:, None],
                               (n, N_GRAPHS))
    return _tc_pool_linear(h, batch2d, W_lin, b_lin[None, :])


# speedup vs baseline: 1.0588x; 1.0588x over previous
import functools

import jax
import jax.numpy as jnp
from jax import lax
from jax.experimental import pallas as pl
from jax.experimental.pallas import tpu as pltpu
from jax.experimental.pallas import tpu_sc as plsc

N_GRAPHS = 128

NC, NS = 2, 16
NW = NC * NS
CHUNK = 128


def _sc_segment_sum(h, src2d, dst2d, n_nodes, agg_rows, ch_per_w):
    d = h.shape[1]
    zc = agg_rows // NS // CHUNK
    out_rows = agg_rows // NS
    gs = 8
    n_gpairs = ch_per_w // (2 * gs)
    n_pairs = gs // 2

    mesh = plsc.VectorSubcoreMesh(core_axis_name="c", subcore_axis_name="s")

    @functools.partial(
        pl.kernel,
        out_type=jax.ShapeDtypeStruct((NC, agg_rows, d), jnp.float32),
        mesh=mesh,
        scratch_types=[
            pltpu.VMEM((gs, CHUNK), jnp.int32),
            pltpu.VMEM((gs, CHUNK), jnp.int32),
            pltpu.VMEM((gs, CHUNK), jnp.int32),
            pltpu.VMEM((gs, CHUNK), jnp.int32),
            pltpu.VMEM((CHUNK, d), jnp.float32),
            pltpu.VMEM((CHUNK, d), jnp.float32),
            pltpu.VMEM_SHARED((agg_rows, d), jnp.float32),
            pltpu.SemaphoreType.DMA,
            pltpu.SemaphoreType.DMA,
            pltpu.SemaphoreType.DMA,
        ],
    )
    def kern(h_hbm, src_hbm, dst_hbm, out_hbm, src_a, dst_a, src_b, dst_b,
             rows_a, rows_b, agg_s, sem_a, sem_b, sem_i):
        cid = lax.axis_index("c")
        sid = lax.axis_index("s")
        wid = sid * NC + cid
        base = wid * ch_per_w

        def stage_start(g, sv, dv):
            pltpu.async_copy(src_hbm.at[pl.ds(base + g * gs, gs)], sv, sem_i)
            pltpu.async_copy(dst_hbm.at[pl.ds(base + g * gs, gs)], dv, sem_i)

        def stage_wait(g, sv, dv):
            pltpu.make_async_copy(
                src_hbm.at[pl.ds(base + g * gs, gs)], sv, sem_i).wait()
            pltpu.make_async_copy(
                dst_hbm.at[pl.ds(base + g * gs, gs)], dv, sem_i).wait()

        with jax.named_scope("agg_zero"):
            def zero_body(i, _):
                rows_a[i // (d // 16), pl.ds((i % (d // 16)) * 16, 16)] = (
                    jnp.zeros((16,), jnp.float32))
                return _
            lax.fori_loop(0, CHUNK * (d // 16), zero_body, None)
            stage_start(0, src_a, dst_a)
            for m in range(zc):
                pltpu.async_copy(
                    rows_a,
                    agg_s.at[pl.ds(sid * (zc * CHUNK) + m * CHUNK, CHUNK)],
                    sem_a)
            for m in range(zc):
                pltpu.make_async_copy(
                    rows_a,
                    agg_s.at[pl.ds(sid * (zc * CHUNK) + m * CHUNK, CHUNK)],
                    sem_a).wait()
            stage_wait(0, src_a, dst_a)

        plsc.subcore_barrier()

        def start(idx, rows, sem):
            pltpu.async_copy(h_hbm.at[idx], rows, sem)

        def wait(idx, rows, sem):
            pltpu.make_async_copy(h_hbm.at[idx], rows, sem).wait()

        def scat(idx, rows):
            pltpu.sync_copy(rows, agg_s.at[idx], add=True)

        def run_group(sv, dv):
            start(sv.at[0], rows_a, sem_a)

            def pair_body(jj, _):
                j0 = 2 * jj
                start(sv.at[j0 + 1], rows_b, sem_b)
                wait(sv.at[j0], rows_a, sem_a)
                scat(dv.at[j0], rows_a)

                @pl.when(jj < n_pairs - 1)
                def _():
                    start(sv.at[j0 + 2], rows_a, sem_a)

                wait(sv.at[j0 + 1], rows_b, sem_b)
                scat(dv.at[j0 + 1], rows_b)
                return _
            lax.fori_loop(0, n_pairs, pair_body, None)

        def gpair_body(gg, _):
            g0 = 2 * gg

            @pl.when(gg > 0)
            def _():
                stage_wait(g0, src_a, dst_a)

            stage_start(g0 + 1, src_b, dst_b)
            run_group(src_a, dst_a)
            stage_wait(g0 + 1, src_b, dst_b)

            @pl.when(gg < n_gpairs - 1)
            def _():
                stage_start(g0 + 2, src_a, dst_a)

            run_group(src_b, dst_b)
            return _
        with jax.named_scope("agg_edges"):
            lax.fori_loop(0, n_gpairs, gpair_body, None)

        plsc.subcore_barrier()

        with jax.named_scope("agg_writeback"):
            pltpu.sync_copy(agg_s.at[pl.ds(sid * out_rows, out_rows)],
                            out_hbm.at[cid, pl.ds(sid * out_rows, out_rows)])

    return kern(h, src2d, dst2d)


def _tc_layer(p, h, w_rel, w_root, bias, relu):
    n, d = h.shape
    blk = 2000
    grid = n // blk

    def body(p_ref, h_ref, wr_ref, wt_ref, b_ref, o_ref):
        agg = p_ref[0] + p_ref[1]
        acc = jnp.dot(agg, wr_ref[...], preferred_element_type=jnp.float32)
        acc = acc + jnp.dot(h_ref[...], wt_ref[...],
                            preferred_element_type=jnp.float32)
        acc = acc + b_ref[...]
        o_ref[...] = jnp.maximum(acc, 0.0) if relu else acc

    row_spec = pl.BlockSpec((blk, d), lambda i: (i, 0))
    full = pl.BlockSpec((d, d), lambda i: (0, 0))
    return pl.pallas_call(
        body,
        grid=(grid,),
        in_specs=[pl.BlockSpec((2, blk, d), lambda i: (0, i, 0)), row_spec,
                  full, full, pl.BlockSpec((1, d), lambda i: (0, 0))],
        out_specs=row_spec,
        out_shape=jax.ShapeDtypeStruct((n, d), jnp.float32),
    )(p, h, w_rel, w_root, bias)


def _tc_pool_linear(h, batch2d, w_lin, b_lin2d):
    n, d = h.shape
    g = N_GRAPHS
    c = w_lin.shape[1]
    blk = 2000
    grid = n // blk

    def body(h_ref, bt_ref, wl_ref, bl_ref, o_ref, acc_ref, cnt_ref):
        i = pl.program_id(0)

        @pl.when(i == 0)
        def _():
            acc_ref[...] = jnp.zeros_like(acc_ref)
            cnt_ref[...] = jnp.zeros_like(cnt_ref)

        bt = bt_ref[...].astype(jnp.int32)
        onehot = (bt == lax.broadcasted_iota(jnp.int32, (blk, g), 1)
                  ).astype(jnp.float32)
        acc_ref[...] += lax.dot_general(
            onehot, h_ref[...], (((0,), (0,)), ((), ())),
            preferred_element_type=jnp.float32)
        cnt_ref[...] += lax.dot_general(
            onehot, jnp.ones((blk, g), jnp.float32), (((0,), (0,)), ((), ())),
            preferred_element_type=jnp.float32)

        @pl.when(i == grid - 1)
        def _():
            pooled = acc_ref[...] / jnp.maximum(cnt_ref[...], 1.0)
            o_ref[...] = jnp.dot(pooled, wl_ref[...],
                                 preferred_element_type=jnp.float32) + bl_ref[...]

    return pl.pallas_call(
        body,
        grid=(grid,),
        in_specs=[
            pl.BlockSpec((blk, d), lambda i: (i, 0)),
            pl.BlockSpec((blk, g), lambda i: (i, 0)),
            pl.BlockSpec((d, c), lambda i: (0, 0)),
            pl.BlockSpec((1, c), lambda i: (0, 0)),
        ],
        out_specs=pl.BlockSpec((g, c), lambda i: (0, 0)),
        out_shape=jax.ShapeDtypeStruct((g, c), jnp.float32),
        scratch_shapes=[
            pltpu.VMEM((g, g), jnp.float32),
            pltpu.VMEM((g, g), jnp.float32),
        ],
    )(h, batch2d, w_lin, b_lin2d)


def kernel(x, edge_index, batch, W_rel, W_root, b, W_lin, b_lin):
    n, d = x.shape
    e = edge_index.shape[1]

    ch_per_w = -(-(-(-e // (NW * CHUNK))) // 16) * 16
    e_pad = NW * CHUNK * ch_per_w
    agg_rows = -(-(n + 1) // (NS * CHUNK)) * (NS * CHUNK)

    src = edge_index[0].astype(jnp.int32)
    dst = edge_index[1].astype(jnp.int32)
    pad = e_pad - e
    pad_src = jnp.arange(pad, dtype=jnp.int32) % n
    pad_dst = n + jnp.arange(pad, dtype=jnp.int32) % (agg_rows - n)
    src2d = jnp.concatenate([src, pad_src]).reshape(-1, CHUNK)
    dst2d = jnp.concatenate([dst, pad_dst]).reshape(-1, CHUNK)

    h = x
    for i in range(8):
        p = _sc_segment_sum(h, src2d, dst2d, n, agg_rows, ch_per_w)
        h = _tc_layer(p, h, W_rel[i], W_root[i], b[i][None, :],
                      relu=(i < 7))

    batch2d = jnp.broadcast_to(batch.astype(jnp.int16)[:, None],
                               (n, N_GRAPHS))
    return _tc_pool_linear(h, batch2d, W_lin, b_lin[None, :])


# speedup vs baseline: 1.0657x; 1.0065x over previous
import functools

import jax
import jax.numpy as jnp
from jax import lax
from jax.experimental import pallas as pl
from jax.experimental.pallas import tpu as pltpu
from jax.experimental.pallas import tpu_sc as plsc

N_GRAPHS = 128

NC, NS = 2, 16
NW = NC * NS
CHUNK = 128


def _sc_segment_sum(h, src2d, dst2d, n_nodes, agg_rows, ch_per_w):
    d = h.shape[1]
    zc = agg_rows // NS // CHUNK
    out_rows = agg_rows // NS
    gs = 8
    n_gpairs = ch_per_w // (2 * gs)
    n_pairs = gs // 2

    mesh = plsc.VectorSubcoreMesh(core_axis_name="c", subcore_axis_name="s")

    @functools.partial(
        pl.kernel,
        out_type=jax.ShapeDtypeStruct((NC, agg_rows, d), jnp.float32),
        mesh=mesh,
        scratch_types=[
            pltpu.VMEM((gs, CHUNK), jnp.int32),
            pltpu.VMEM((gs, CHUNK), jnp.int32),
            pltpu.VMEM((gs, CHUNK), jnp.int32),
            pltpu.VMEM((gs, CHUNK), jnp.int32),
            pltpu.VMEM((CHUNK, d), jnp.float32),
            pltpu.VMEM((CHUNK, d), jnp.float32),
            pltpu.VMEM_SHARED((agg_rows, d), jnp.float32),
            pltpu.SemaphoreType.DMA,
            pltpu.SemaphoreType.DMA,
            pltpu.SemaphoreType.DMA,
        ],
    )
    def kern(h_hbm, src_hbm, dst_hbm, out_hbm, src_a, dst_a, src_b, dst_b,
             rows_a, rows_b, agg_s, sem_a, sem_b, sem_i):
        cid = lax.axis_index("c")
        sid = lax.axis_index("s")
        wid = sid * NC + cid
        base = wid * ch_per_w

        def stage_start(g, sv, dv):
            pltpu.async_copy(src_hbm.at[pl.ds(base + g * gs, gs)], sv, sem_i)
            pltpu.async_copy(dst_hbm.at[pl.ds(base + g * gs, gs)], dv, sem_i)

        def stage_wait(g, sv, dv):
            pltpu.make_async_copy(
                src_hbm.at[pl.ds(base + g * gs, gs)], sv, sem_i).wait()
            pltpu.make_async_copy(
                dst_hbm.at[pl.ds(base + g * gs, gs)], dv, sem_i).wait()

        with jax.named_scope("agg_zero"):
            def zero_body(i, _):
                rows_a[i // (d // 16), pl.ds((i % (d // 16)) * 16, 16)] = (
                    jnp.zeros((16,), jnp.float32))
                return _
            lax.fori_loop(0, CHUNK * (d // 16), zero_body, None)
            stage_start(0, src_a, dst_a)
            for m in range(zc):
                pltpu.async_copy(
                    rows_a,
                    agg_s.at[pl.ds(sid * (zc * CHUNK) + m * CHUNK, CHUNK)],
                    sem_a)
            for m in range(zc):
                pltpu.make_async_copy(
                    rows_a,
                    agg_s.at[pl.ds(sid * (zc * CHUNK) + m * CHUNK, CHUNK)],
                    sem_a).wait()
            stage_wait(0, src_a, dst_a)

        plsc.subcore_barrier()

        def start(idx, rows, sem):
            pltpu.async_copy(h_hbm.at[idx], rows, sem)

        def wait(idx, rows, sem):
            pltpu.make_async_copy(h_hbm.at[idx], rows, sem).wait()

        def scat(idx, rows):
            pltpu.sync_copy(rows, agg_s.at[idx], add=True)

        def run_group(sv, dv):
            start(sv.at[0], rows_a, sem_a)

            def pair_body(jj, _):
                j0 = 2 * jj
                start(sv.at[j0 + 1], rows_b, sem_b)
                wait(sv.at[j0], rows_a, sem_a)
                scat(dv.at[j0], rows_a)

                @pl.when(jj < n_pairs - 1)
                def _():
                    start(sv.at[j0 + 2], rows_a, sem_a)

                wait(sv.at[j0 + 1], rows_b, sem_b)
                scat(dv.at[j0 + 1], rows_b)
                return _
            lax.fori_loop(0, n_pairs, pair_body, None)

        def gpair_body(gg, _):
            g0 = 2 * gg

            @pl.when(gg > 0)
            def _():
                stage_wait(g0, src_a, dst_a)

            stage_start(g0 + 1, src_b, dst_b)
            run_group(src_a, dst_a)
            stage_wait(g0 + 1, src_b, dst_b)

            @pl.when(gg < n_gpairs - 1)
            def _():
                stage_start(g0 + 2, src_a, dst_a)

            run_group(src_b, dst_b)
            return _
        with jax.named_scope("agg_edges"):
            lax.fori_loop(0, n_gpairs, gpair_body, None)

        plsc.subcore_barrier()

        with jax.named_scope("agg_writeback"):
            pltpu.sync_copy(agg_s.at[pl.ds(sid * out_rows, out_rows)],
                            out_hbm.at[cid, pl.ds(sid * out_rows, out_rows)])

    return kern(h, src2d, dst2d)


def _tc_layer(p, h, w_rel, w_root, bias, relu):
    n, d = h.shape
    blk = 2000
    grid = n // blk

    def body(p_ref, h_ref, wr_ref, wt_ref, b_ref, o_ref):
        agg = p_ref[0] + p_ref[1]
        acc = jnp.dot(agg, wr_ref[...], preferred_element_type=jnp.float32)
        acc = acc + jnp.dot(h_ref[...], wt_ref[...],
                            preferred_element_type=jnp.float32)
        acc = acc + b_ref[...]
        o_ref[...] = jnp.maximum(acc, 0.0) if relu else acc

    row_spec = pl.BlockSpec((blk, d), lambda i: (i, 0))
    full = pl.BlockSpec((d, d), lambda i: (0, 0))
    return pl.pallas_call(
        body,
        grid=(grid,),
        in_specs=[pl.BlockSpec((2, blk, d), lambda i: (0, i, 0)), row_spec,
                  full, full, pl.BlockSpec((1, d), lambda i: (0, 0))],
        out_specs=row_spec,
        out_shape=jax.ShapeDtypeStruct((n, d), jnp.float32),
    )(p, h, w_rel, w_root, bias)


def _tc_last_layer_pool_linear(p, h, w_rel, w_root, bias, batch_col,
                               w_lin, b_lin2d):
    n, d = h.shape
    g = N_GRAPHS
    c = w_lin.shape[1]
    blk = 2000
    grid = n // blk

    def body(p_ref, h_ref, wr_ref, wt_ref, b_ref, bt_ref, wl_ref, bl_ref,
             o_ref, acc_ref, cnt_ref):
        i = pl.program_id(0)

        @pl.when(i == 0)
        def _():
            acc_ref[...] = jnp.zeros_like(acc_ref)
            cnt_ref[...] = jnp.zeros_like(cnt_ref)

        agg = p_ref[0] + p_ref[1]
        hb = jnp.dot(agg, wr_ref[...], preferred_element_type=jnp.float32)
        hb = hb + jnp.dot(h_ref[...], wt_ref[...],
                          preferred_element_type=jnp.float32)
        hb = hb + b_ref[...]

        bt = bt_ref[...].astype(jnp.int32)
        onehot = (bt == lax.broadcasted_iota(jnp.int32, (blk, g), 1)
                  ).astype(jnp.float32)
        acc_ref[...] += lax.dot_general(
            onehot, hb, (((0,), (0,)), ((), ())),
            preferred_element_type=jnp.float32)
        cnt_ref[...] += lax.dot_general(
            onehot, jnp.ones((blk, g), jnp.float32), (((0,), (0,)), ((), ())),
            preferred_element_type=jnp.float32)

        @pl.when(i == grid - 1)
        def _():
            pooled = acc_ref[...] / jnp.maximum(cnt_ref[...], 1.0)
            o_ref[...] = jnp.dot(pooled, wl_ref[...],
                                 preferred_element_type=jnp.float32) + bl_ref[...]

    return pl.pallas_call(
        body,
        grid=(grid,),
        in_specs=[
            pl.BlockSpec((2, blk, d), lambda i: (0, i, 0)),
            pl.BlockSpec((blk, d), lambda i: (i, 0)),
            pl.BlockSpec((d, d), lambda i: (0, 0)),
            pl.BlockSpec((d, d), lambda i: (0, 0)),
            pl.BlockSpec((1, d), lambda i: (0, 0)),
            pl.BlockSpec((blk, 1), lambda i: (i, 0)),
            pl.BlockSpec((d, c), lambda i: (0, 0)),
            pl.BlockSpec((1, c), lambda i: (0, 0)),
        ],
        out_specs=pl.BlockSpec((g, c), lambda i: (0, 0)),
        out_shape=jax.ShapeDtypeStruct((g, c), jnp.float32),
        scratch_shapes=[
            pltpu.VMEM((g, g), jnp.float32),
            pltpu.VMEM((g, g), jnp.float32),
        ],
    )(p, h, w_rel, w_root, bias, batch_col, w_lin, b_lin2d)


def kernel(x, edge_index, batch, W_rel, W_root, b, W_lin, b_lin):
    n, d = x.shape
    e = edge_index.shape[1]

    ch_per_w = -(-(-(-e // (NW * CHUNK))) // 16) * 16
    e_pad = NW * CHUNK * ch_per_w
    agg_rows = -(-(n + 1) // (NS * CHUNK)) * (NS * CHUNK)

    src = edge_index[0].astype(jnp.int32)
    dst = edge_index[1].astype(jnp.int32)
    pad = e_pad - e
    pad_src = jnp.arange(pad, dtype=jnp.int32) % n
    pad_dst = n + jnp.arange(pad, dtype=jnp.int32) % (agg_rows - n)
    src2d = jnp.concatenate([src, pad_src]).reshape(-1, CHUNK)
    dst2d = jnp.concatenate([dst, pad_dst]).reshape(-1, CHUNK)

    h = x
    for i in range(7):
        p = _sc_segment_sum(h, src2d, dst2d, n, agg_rows, ch_per_w)
        h = _tc_layer(p, h, W_rel[i], W_root[i], b[i][None, :], relu=True)

    p = _sc_segment_sum(h, src2d, dst2d, n, agg_rows, ch_per_w)
    batch_col = batch.astype(jnp.int32)[:, None]
    return _tc_last_layer_pool_linear(p, h, W_rel[7], W_root[7],
                                      b[7][None, :], batch_col,
                                      W_lin, b_lin[None, :])


# speedup vs baseline: 1.0767x; 1.0103x over previous
import functools

import jax
import jax.numpy as jnp
from jax import lax
from jax.experimental import pallas as pl
from jax.experimental.pallas import tpu as pltpu
from jax.experimental.pallas import tpu_sc as plsc

N_GRAPHS = 128

NC, NS = 2, 16
NW = NC * NS
CHUNK = 128


def _sc_segment_sum(h, src2d, dst2d, n_nodes, agg_rows, ch_per_w):
    d = h.shape[1]
    zc = agg_rows // NS // CHUNK
    out_rows = agg_rows // NS
    gs = 8
    n_gpairs = ch_per_w // (2 * gs)
    n_pairs = gs // 2

    mesh = plsc.VectorSubcoreMesh(core_axis_name="c", subcore_axis_name="s")

    @functools.partial(
        pl.kernel,
        out_type=jax.ShapeDtypeStruct((NC, agg_rows, d), jnp.float32),
        mesh=mesh,
        scratch_types=[
            pltpu.VMEM((gs, CHUNK), jnp.int32),
            pltpu.VMEM((gs, CHUNK), jnp.int32),
            pltpu.VMEM((gs, CHUNK), jnp.int32),
            pltpu.VMEM((gs, CHUNK), jnp.int32),
            pltpu.VMEM((CHUNK, d), jnp.float32),
            pltpu.VMEM((CHUNK, d), jnp.float32),
            pltpu.VMEM_SHARED((agg_rows, d), jnp.float32),
            pltpu.SemaphoreType.DMA,
            pltpu.SemaphoreType.DMA,
            pltpu.SemaphoreType.DMA,
            pltpu.SemaphoreType.DMA,
        ],
    )
    def kern(h_hbm, src_hbm, dst_hbm, out_hbm, src_a, dst_a, src_b, dst_b,
             rows_a, rows_b, agg_s, sem_a, sem_b, sem_i, sem_z):
        cid = lax.axis_index("c")
        sid = lax.axis_index("s")
        wid = sid * NC + cid
        base = wid * ch_per_w

        def stage_start(g, sv, dv):
            pltpu.async_copy(src_hbm.at[pl.ds(base + g * gs, gs)], sv, sem_i)
            pltpu.async_copy(dst_hbm.at[pl.ds(base + g * gs, gs)], dv, sem_i)

        def stage_wait(g, sv, dv):
            pltpu.make_async_copy(
                src_hbm.at[pl.ds(base + g * gs, gs)], sv, sem_i).wait()
            pltpu.make_async_copy(
                dst_hbm.at[pl.ds(base + g * gs, gs)], dv, sem_i).wait()

        def start(idx, rows, sem):
            pltpu.async_copy(h_hbm.at[idx], rows, sem)

        def wait(idx, rows, sem):
            pltpu.make_async_copy(h_hbm.at[idx], rows, sem).wait()

        def scat(idx, rows):
            pltpu.sync_copy(rows, agg_s.at[idx], add=True)

        with jax.named_scope("agg_zero"):
            def zero_body(i, _):
                rows_b[i // (d // 16), pl.ds((i % (d // 16)) * 16, 16)] = (
                    jnp.zeros((16,), jnp.float32))
                return _
            lax.fori_loop(0, CHUNK * (d // 16), zero_body, None)
            stage_start(0, src_a, dst_a)
            for m in range(zc):
                pltpu.async_copy(
                    rows_b,
                    agg_s.at[pl.ds(sid * (zc * CHUNK) + m * CHUNK, CHUNK)],
                    sem_z)
            stage_wait(0, src_a, dst_a)
            start(src_a.at[0], rows_a, sem_a)
            for m in range(zc):
                pltpu.make_async_copy(
                    rows_b,
                    agg_s.at[pl.ds(sid * (zc * CHUNK) + m * CHUNK, CHUNK)],
                    sem_z).wait()

        plsc.subcore_barrier()

        def run_group(sv, dv, skip_first_start=None):
            if skip_first_start is None:
                start(sv.at[0], rows_a, sem_a)
            else:
                @pl.when(jnp.logical_not(skip_first_start))
                def _():
                    start(sv.at[0], rows_a, sem_a)

            def pair_body(jj, _):
                j0 = 2 * jj
                start(sv.at[j0 + 1], rows_b, sem_b)
                wait(sv.at[j0], rows_a, sem_a)
                scat(dv.at[j0], rows_a)

                @pl.when(jj < n_pairs - 1)
                def _():
                    start(sv.at[j0 + 2], rows_a, sem_a)

                wait(sv.at[j0 + 1], rows_b, sem_b)
                scat(dv.at[j0 + 1], rows_b)
                return _
            lax.fori_loop(0, n_pairs, pair_body, None)

        def gpair_body(gg, _):
            g0 = 2 * gg

            @pl.when(gg > 0)
            def _():
                stage_wait(g0, src_a, dst_a)

            stage_start(g0 + 1, src_b, dst_b)
            run_group(src_a, dst_a, skip_first_start=(gg == 0))
            stage_wait(g0 + 1, src_b, dst_b)

            @pl.when(gg < n_gpairs - 1)
            def _():
                stage_start(g0 + 2, src_a, dst_a)

            run_group(src_b, dst_b)
            return _
        with jax.named_scope("agg_edges"):
            lax.fori_loop(0, n_gpairs, gpair_body, None)

        plsc.subcore_barrier()

        with jax.named_scope("agg_writeback"):
            pltpu.sync_copy(agg_s.at[pl.ds(sid * out_rows, out_rows)],
                            out_hbm.at[cid, pl.ds(sid * out_rows, out_rows)])

    return kern(h, src2d, dst2d)


def _tc_layer(p, h, w_rel, w_root, bias, relu):
    n, d = h.shape
    blk = 2000
    grid = n // blk

    def body(p_ref, h_ref, wr_ref, wt_ref, b_ref, o_ref):
        agg = p_ref[0] + p_ref[1]
        acc = jnp.dot(agg, wr_ref[...], preferred_element_type=jnp.float32)
        acc = acc + jnp.dot(h_ref[...], wt_ref[...],
                            preferred_element_type=jnp.float32)
        acc = acc + b_ref[...]
        o_ref[...] = jnp.maximum(acc, 0.0) if relu else acc

    row_spec = pl.BlockSpec((blk, d), lambda i: (i, 0))
    full = pl.BlockSpec((d, d), lambda i: (0, 0))
    return pl.pallas_call(
        body,
        grid=(grid,),
        in_specs=[pl.BlockSpec((2, blk, d), lambda i: (0, i, 0)), row_spec,
                  full, full, pl.BlockSpec((1, d), lambda i: (0, 0))],
        out_specs=row_spec,
        out_shape=jax.ShapeDtypeStruct((n, d), jnp.float32),
    )(p, h, w_rel, w_root, bias)


def _tc_last_layer_pool_linear(p, h, w_rel, w_root, bias, batch_col,
                               w_lin, b_lin2d):
    n, d = h.shape
    g = N_GRAPHS
    c = w_lin.shape[1]
    blk = 2000
    grid = n // blk

    def body(p_ref, h_ref, wr_ref, wt_ref, b_ref, bt_ref, wl_ref, bl_ref,
             o_ref, acc_ref, cnt_ref):
        i = pl.program_id(0)

        @pl.when(i == 0)
        def _():
            acc_ref[...] = jnp.zeros_like(acc_ref)
            cnt_ref[...] = jnp.zeros_like(cnt_ref)

        agg = p_ref[0] + p_ref[1]
        hb = jnp.dot(agg, wr_ref[...], preferred_element_type=jnp.float32)
        hb = hb + jnp.dot(h_ref[...], wt_ref[...],
                          preferred_element_type=jnp.float32)
        hb = hb + b_ref[...]

        bt = bt_ref[...].astype(jnp.int32)
        onehot = (bt == lax.broadcasted_iota(jnp.int32, (blk, g), 1)
                  ).astype(jnp.float32)
        acc_ref[...] += lax.dot_general(
            onehot, hb, (((0,), (0,)), ((), ())),
            preferred_element_type=jnp.float32)
        cnt_ref[...] += lax.dot_general(
            onehot, jnp.ones((blk, g), jnp.float32), (((0,), (0,)), ((), ())),
            preferred_element_type=jnp.float32)

        @pl.when(i == grid - 1)
        def _():
            pooled = acc_ref[...] / jnp.maximum(cnt_ref[...], 1.0)
            o_ref[...] = jnp.dot(pooled, wl_ref[...],
                                 preferred_element_type=jnp.float32) + bl_ref[...]

    return pl.pallas_call(
        body,
        grid=(grid,),
        in_specs=[
            pl.BlockSpec((2, blk, d), lambda i: (0, i, 0)),
            pl.BlockSpec((blk, d), lambda i: (i, 0)),
            pl.BlockSpec((d, d), lambda i: (0, 0)),
            pl.BlockSpec((d, d), lambda i: (0, 0)),
            pl.BlockSpec((1, d), lambda i: (0, 0)),
            pl.BlockSpec((blk, 1), lambda i: (i, 0)),
            pl.BlockSpec((d, c), lambda i: (0, 0)),
            pl.BlockSpec((1, c), lambda i: (0, 0)),
        ],
        out_specs=pl.BlockSpec((g, c), lambda i: (0, 0)),
        out_shape=jax.ShapeDtypeStruct((g, c), jnp.float32),
        scratch_shapes=[
            pltpu.VMEM((g, g), jnp.float32),
            pltpu.VMEM((g, g), jnp.float32),
        ],
    )(p, h, w_rel, w_root, bias, batch_col, w_lin, b_lin2d)


def kernel(x, edge_index, batch, W_rel, W_root, b, W_lin, b_lin):
    n, d = x.shape
    e = edge_index.shape[1]

    ch_per_w = -(-(-(-e // (NW * CHUNK))) // 16) * 16
    e_pad = NW * CHUNK * ch_per_w
    agg_rows = -(-(n + 1) // (NS * CHUNK)) * (NS * CHUNK)

    src = edge_index[0].astype(jnp.int32)
    dst = edge_index[1].astype(jnp.int32)
    pad = e_pad - e
    pad_src = jnp.arange(pad, dtype=jnp.int32) % n
    pad_dst = n + jnp.arange(pad, dtype=jnp.int32) % (agg_rows - n)
    src2d = jnp.concatenate([src, pad_src]).reshape(-1, CHUNK)
    dst2d = jnp.concatenate([dst, pad_dst]).reshape(-1, CHUNK)

    h = x
    for i in range(7):
        p = _sc_segment_sum(h, src2d, dst2d, n, agg_rows, ch_per_w)
        h = _tc_layer(p, h, W_rel[i], W_root[i], b[i][None, :], relu=True)

    p = _sc_segment_sum(h, src2d, dst2d, n, agg_rows, ch_per_w)
    batch_col = batch.astype(jnp.int32)[:, None]
    return _tc_last_layer_pool_linear(p, h, W_rel[7], W_root[7],
                                      b[7][None, :], batch_col,
                                      W_lin, b_lin[None, :])


# speedup vs baseline: 1.0791x; 1.0022x over previous
import functools

import jax
import jax.numpy as jnp
from jax import lax
from jax.experimental import pallas as pl
from jax.experimental.pallas import tpu as pltpu
from jax.experimental.pallas import tpu_sc as plsc

N_GRAPHS = 128

NC, NS = 2, 16
NW = NC * NS
CHUNK = 128


def _sc_segment_sum(h, src2d, dst2d, n_nodes, agg_rows, ch_per_w):
    d = h.shape[1]
    zc = agg_rows // NS // CHUNK
    out_rows = agg_rows // NS
    gs = 8
    n_gpairs = ch_per_w // (2 * gs)
    n_pairs = gs // 2

    mesh = plsc.VectorSubcoreMesh(core_axis_name="c", subcore_axis_name="s")

    @functools.partial(
        pl.kernel,
        out_type=jax.ShapeDtypeStruct((NC, agg_rows, d), jnp.float32),
        mesh=mesh,
        scratch_types=[
            pltpu.VMEM((gs, CHUNK), jnp.int32),
            pltpu.VMEM((gs, CHUNK), jnp.int32),
            pltpu.VMEM((gs, CHUNK), jnp.int32),
            pltpu.VMEM((gs, CHUNK), jnp.int32),
            pltpu.VMEM((CHUNK, d), jnp.float32),
            pltpu.VMEM((CHUNK, d), jnp.float32),
            pltpu.VMEM_SHARED((agg_rows, d), jnp.float32),
            pltpu.SemaphoreType.DMA,
            pltpu.SemaphoreType.DMA,
            pltpu.SemaphoreType.DMA,
            pltpu.SemaphoreType.DMA,
        ],
    )
    def kern(h_hbm, src_hbm, dst_hbm, out_hbm, src_a, dst_a, src_b, dst_b,
             rows_a, rows_b, agg_s, sem_a, sem_b, sem_i, sem_z):
        cid = lax.axis_index("c")
        sid = lax.axis_index("s")
        wid = sid * NC + cid
        base = wid * ch_per_w

        def stage_start(g, sv, dv):
            pltpu.async_copy(src_hbm.at[pl.ds(base + g * gs, gs)], sv, sem_i)
            pltpu.async_copy(dst_hbm.at[pl.ds(base + g * gs, gs)], dv, sem_i)

        def stage_wait(g, sv, dv):
            pltpu.make_async_copy(
                src_hbm.at[pl.ds(base + g * gs, gs)], sv, sem_i).wait()
            pltpu.make_async_copy(
                dst_hbm.at[pl.ds(base + g * gs, gs)], dv, sem_i).wait()

        def start(idx, rows, sem):
            pltpu.async_copy(h_hbm.at[idx], rows, sem)

        def wait(idx, rows, sem):
            pltpu.make_async_copy(h_hbm.at[idx], rows, sem).wait()

        def scat(idx, rows):
            pltpu.sync_copy(rows, agg_s.at[idx], add=True)

        with jax.named_scope("agg_zero"):
            def zero_body(i, _):
                rows_b[i // (d // 16), pl.ds((i % (d // 16)) * 16, 16)] = (
                    jnp.zeros((16,), jnp.float32))
                return _
            lax.fori_loop(0, CHUNK * (d // 16), zero_body, None)
            stage_start(0, src_a, dst_a)
            for m in range(zc):
                pltpu.async_copy(
                    rows_b,
                    agg_s.at[pl.ds(sid * (zc * CHUNK) + m * CHUNK, CHUNK)],
                    sem_z)
            stage_wait(0, src_a, dst_a)
            start(src_a.at[0], rows_a, sem_a)
            for m in range(zc):
                pltpu.make_async_copy(
                    rows_b,
                    agg_s.at[pl.ds(sid * (zc * CHUNK) + m * CHUNK, CHUNK)],
                    sem_z).wait()

        plsc.subcore_barrier()

        def run_group(sv, dv, skip_first_start=None):
            if skip_first_start is None:
                start(sv.at[0], rows_a, sem_a)
            else:
                @pl.when(jnp.logical_not(skip_first_start))
                def _():
                    start(sv.at[0], rows_a, sem_a)

            def pair_body(jj, _):
                j0 = 2 * jj
                start(sv.at[j0 + 1], rows_b, sem_b)
                wait(sv.at[j0], rows_a, sem_a)
                scat(dv.at[j0], rows_a)

                @pl.when(jj < n_pairs - 1)
                def _():
                    start(sv.at[j0 + 2], rows_a, sem_a)

                wait(sv.at[j0 + 1], rows_b, sem_b)
                scat(dv.at[j0 + 1], rows_b)
                return _
            lax.fori_loop(0, n_pairs, pair_body, None)

        def gpair_body(gg, _):
            g0 = 2 * gg

            @pl.when(gg > 0)
            def _():
                stage_wait(g0, src_a, dst_a)

            stage_start(g0 + 1, src_b, dst_b)
            run_group(src_a, dst_a, skip_first_start=(gg == 0))
            stage_wait(g0 + 1, src_b, dst_b)

            @pl.when(gg < n_gpairs - 1)
            def _():
                stage_start(g0 + 2, src_a, dst_a)

            run_group(src_b, dst_b)
            return _
        with jax.named_scope("agg_edges"):
            lax.fori_loop(0, n_gpairs, gpair_body, None)

        plsc.subcore_barrier()

        with jax.named_scope("agg_writeback"):
            pltpu.sync_copy(agg_s.at[pl.ds(sid * out_rows, out_rows)],
                            out_hbm.at[cid, pl.ds(sid * out_rows, out_rows)])

    return kern(h, src2d, dst2d)


def _tc_layer(p, h, w_rel, w_root, bias, li, relu):
    n, d = h.shape
    blk = 2000
    grid = n // blk

    def body(p_ref, h_ref, wr_ref, wt_ref, b_ref, o_ref):
        agg = p_ref[0] + p_ref[1]
        acc = jnp.dot(agg, wr_ref[0], preferred_element_type=jnp.float32)
        acc = acc + jnp.dot(h_ref[...], wt_ref[0],
                            preferred_element_type=jnp.float32)
        acc = acc + b_ref[li][None, :]
        o_ref[...] = jnp.maximum(acc, 0.0) if relu else acc

    row_spec = pl.BlockSpec((blk, d), lambda i: (i, 0))
    wsel = pl.BlockSpec((1, d, d), lambda i: (li, 0, 0))
    return pl.pallas_call(
        body,
        grid=(grid,),
        in_specs=[pl.BlockSpec((2, blk, d), lambda i: (0, i, 0)), row_spec,
                  wsel, wsel, pl.BlockSpec((8, d), lambda i: (0, 0))],
        out_specs=row_spec,
        out_shape=jax.ShapeDtypeStruct((n, d), jnp.float32),
    )(p, h, w_rel, w_root, bias)


def _tc_last_layer_pool_linear(p, h, w_rel, w_root, bias, batch_col,
                               w_lin, b_lin2d):
    n, d = h.shape
    g = N_GRAPHS
    c = w_lin.shape[1]
    blk = 2000
    grid = n // blk

    def body(p_ref, h_ref, wr_ref, wt_ref, b_ref, bt_ref, wl_ref, bl_ref,
             o_ref, acc_ref, cnt_ref):
        i = pl.program_id(0)

        @pl.when(i == 0)
        def _():
            acc_ref[...] = jnp.zeros_like(acc_ref)
            cnt_ref[...] = jnp.zeros_like(cnt_ref)

        agg = p_ref[0] + p_ref[1]
        hb = jnp.dot(agg, wr_ref[0], preferred_element_type=jnp.float32)
        hb = hb + jnp.dot(h_ref[...], wt_ref[0],
                          preferred_element_type=jnp.float32)
        hb = hb + b_ref[7][None, :]

        bt = bt_ref[...].astype(jnp.int32)
        onehot = (bt == lax.broadcasted_iota(jnp.int32, (blk, g), 1)
                  ).astype(jnp.float32)
        acc_ref[...] += lax.dot_general(
            onehot, hb, (((0,), (0,)), ((), ())),
            preferred_element_type=jnp.float32)
        cnt_ref[...] += lax.dot_general(
            onehot, jnp.ones((blk, g), jnp.float32), (((0,), (0,)), ((), ())),
            preferred_element_type=jnp.float32)

        @pl.when(i == grid - 1)
        def _():
            pooled = acc_ref[...] / jnp.maximum(cnt_ref[...], 1.0)
            o_ref[...] = jnp.dot(pooled, wl_ref[...],
                                 preferred_element_type=jnp.float32) + bl_ref[...]

    return pl.pallas_call(
        body,
        grid=(grid,),
        in_specs=[
            pl.BlockSpec((2, blk, d), lambda i: (0, i, 0)),
            pl.BlockSpec((blk, d), lambda i: (i, 0)),
            pl.BlockSpec((1, d, d), lambda i: (7, 0, 0)),
            pl.BlockSpec((1, d, d), lambda i: (7, 0, 0)),
            pl.BlockSpec((8, d), lambda i: (0, 0)),
            pl.BlockSpec((blk, 1), lambda i: (i, 0)),
            pl.BlockSpec((d, c), lambda i: (0, 0)),
            pl.BlockSpec((1, c), lambda i: (0, 0)),
        ],
        out_specs=pl.BlockSpec((g, c), lambda i: (0, 0)),
        out_shape=jax.ShapeDtypeStruct((g, c), jnp.float32),
        scratch_shapes=[
            pltpu.VMEM((g, g), jnp.float32),
            pltpu.VMEM((g, g), jnp.float32),
        ],
    )(p, h, w_rel, w_root, bias, batch_col, w_lin, b_lin2d)


def kernel(x, edge_index, batch, W_rel, W_root, b, W_lin, b_lin):
    n, d = x.shape
    e = edge_index.shape[1]

    ch_per_w = -(-(-(-e // (NW * CHUNK))) // 16) * 16
    e_pad = NW * CHUNK * ch_per_w
    agg_rows = -(-(n + 1) // (NS * CHUNK)) * (NS * CHUNK)

    src = edge_index[0].astype(jnp.int32)
    dst = edge_index[1].astype(jnp.int32)
    pad = e_pad - e
    pad_src = jnp.arange(pad, dtype=jnp.int32) % n
    pad_dst = n + jnp.arange(pad, dtype=jnp.int32) % (agg_rows - n)
    src2d = jnp.concatenate([src, pad_src]).reshape(-1, CHUNK)
    dst2d = jnp.concatenate([dst, pad_dst]).reshape(-1, CHUNK)

    h = x
    for i in range(7):
        p = _sc_segment_sum(h, src2d, dst2d, n, agg_rows, ch_per_w)
        h = _tc_layer(p, h, W_rel, W_root, b, i, relu=True)

    p = _sc_segment_sum(h, src2d, dst2d, n, agg_rows, ch_per_w)
    batch_col = batch.astype(jnp.int32)[:, None]
    return _tc_last_layer_pool_linear(p, h, W_rel, W_root, b, batch_col,
                                      W_lin, b_lin[None, :])
